# baseline (speedup 1.0000x reference)
import functools

import jax
import jax.numpy as jnp
from jax import lax
from jax.experimental import pallas as pl
from jax.experimental.pallas import tpu as pltpu
from jax.experimental.pallas import tpu_sc as plsc

_LANES = 16
_TM = 512


def _mm_body(a_ref, b_ref, o_ref):
    o_ref[...] = lax.dot_general(
        a_ref[...], b_ref[...], (((1,), (1,)), ((), ())),
        preferred_element_type=jnp.float32).astype(jnp.bfloat16)


@functools.lru_cache(maxsize=None)
def _make_gram(n_pad: int, d: int):
    grid = (n_pad // _TM, n_pad // _TM)
    return pl.pallas_call(
        _mm_body,
        grid=grid,
        in_specs=[pl.BlockSpec((_TM, d), lambda i, j: (i, 0)),
                  pl.BlockSpec((_TM, d), lambda i, j: (j, 0))],
        out_specs=pl.BlockSpec((_TM, _TM), lambda i, j: (i, j)),
        out_shape=jax.ShapeDtypeStruct((n_pad, n_pad), jnp.bfloat16),
    )


@functools.lru_cache(maxsize=None)
def _make_decoder(n_nodes: int, n_pad: int, n_edges: int):
    info = plsc.get_sparse_core_info()
    nw = info.num_cores * info.num_subcores
    assert n_edges % nw == 0
    per_w = n_edges // nw
    chunk = 0
    for c in range(128, 15, -16):
        if per_w % c == 0:
            chunk = c
            break
    assert chunk > 0
    n_chunks = per_w // chunk
    n_rows = n_pad * n_pad // 32

    mesh = plsc.VectorSubcoreMesh(core_axis_name="c", subcore_axis_name="s")
    nbuf = 8

    @functools.partial(
        pl.kernel,
        out_type=jax.ShapeDtypeStruct((n_edges,), jnp.float32),
        mesh=mesh,
        compiler_params=pltpu.CompilerParams(needs_layout_passes=False,
                                             use_tc_tiling_on_sc=False),
        scratch_types=[
            pltpu.VMEM((n_chunks, chunk), jnp.int32),
            pltpu.VMEM((n_chunks, chunk), jnp.int32),
            pltpu.VMEM((nbuf, chunk), jnp.int32),
            pltpu.VMEM((nbuf, chunk, 16), jnp.int32),
            pltpu.VMEM((per_w,), jnp.float32),
        ] + [pltpu.SemaphoreType.DMA] * nbuf,
    )
    def decode(gw_hbm, ei_hbm, out_hbm, idx_s, idx_d, row_buf, w_buf,
               out_buf, *sems):
        wid = lax.axis_index("s") * info.num_cores + lax.axis_index("c")
        base = wid * per_w

        pltpu.sync_copy(ei_hbm.at[0, wid], idx_s)
        pltpu.sync_copy(ei_hbm.at[1, wid], idx_d)

        lane = lax.iota(jnp.int32, 16)

        def flat_ids(c, g):
            s_vec = idx_s[c, pl.ds(g * _LANES, _LANES)]
            d_vec = idx_d[c, pl.ds(g * _LANES, _LANES)]
            return s_vec * n_pad + d_vec

        def rowidx(c, slot):
            def g_body(g, _):
                flat = flat_ids(c, g)
                row_buf[slot, pl.ds(g * _LANES, _LANES)] = \
                    lax.shift_right_logical(flat, 5)
                return 0

            lax.fori_loop(0, chunk // _LANES, g_body, 0)

        def fire(c, slot):
            rowidx(c, slot)
            pltpu.make_async_copy(
                gw_hbm.at[row_buf.at[slot]], w_buf.at[slot],
                sems[slot]).start()

        def drain(slot):
            pltpu.make_async_copy(
                gw_hbm.at[row_buf.at[slot]], w_buf.at[slot],
                sems[slot]).wait()

        def compute(c, slot):
            out_base = c * chunk

            def g_body(g, _):
                e_vec = g * _LANES + lane
                flat = flat_ids(c, g)
                wv = lax.shift_right_logical(flat, 1) & 15
                w = plsc.load_gather(w_buf.at[slot], [e_vec, wv])
                lo, hi = plsc.unpack(
                    plsc.bitcast(w, jnp.bfloat16),
                    format=plsc.PackFormat.INTERLEAVED,
                    preferred_element_type=jnp.float32)
                val = jnp.where((flat & 1) == 1, hi, lo)
                res = 1.0 / (1.0 + jnp.exp(-val))
                out_buf[pl.ds(out_base + g * _LANES, _LANES)] = res
                return 0

            lax.fori_loop(0, chunk // _LANES, g_body, 0)

        for s in range(nbuf - 1):
            fire(s, s)

        def pipe_body(i, _):
            for j in range(nbuf):
                c = i * nbuf + j

                @pl.when(c + nbuf - 1 < n_chunks)
                def _():
                    fire(c + nbuf - 1, (j + nbuf - 1) % nbuf)

                drain(j)
                compute(c, j)
            return 0

        lax.fori_loop(0, n_chunks // nbuf, pipe_body, 0)
        for j in range(n_chunks % nbuf):
            c = n_chunks - n_chunks % nbuf + j
            drain(c % nbuf)
            compute(c, c % nbuf)

        pltpu.sync_copy(out_buf, out_hbm.at[pl.ds(base, per_w)])

    return decode, nw, n_chunks, chunk, n_rows


def kernel(z, edge_index):
    n_nodes, d = z.shape
    n_edges = edge_index.shape[1]
    n_pad = -(-n_nodes // _TM) * _TM
    decode, nw, n_chunks, chunk, n_rows = _make_decoder(
        n_nodes, n_pad, n_edges)
    z_bf = jnp.pad(z.astype(jnp.bfloat16), ((0, n_pad - n_nodes), (0, 0)))
    gram = _make_gram(n_pad, d)(z_bf, z_bf)
    gw = lax.bitcast_convert_type(
        gram.reshape(n_rows, 16, 2), jnp.int32)
    ei = edge_index.astype(jnp.int32).reshape(2, nw, n_chunks, chunk)
    return decode(gw, ei)


# speedup vs baseline: 245.4581x; 245.4581x over previous
import functools

import jax
import jax.numpy as jnp
from jax import lax
from jax.experimental import pallas as pl
from jax.experimental.pallas import tpu as pltpu
from jax.experimental.pallas import tpu_sc as plsc

_LANES = 16
_TM = 512


def _mm_body(a_ref, b_ref, o_ref):
    a = a_ref[...]
    for jj in range(_TM // 128):
        p = lax.dot_general(
            a, b_ref[pl.ds(jj * 128, 128), :], (((1,), (1,)), ((), ())),
            preferred_element_type=jnp.float32)
        o_ref[:, jj] = p.reshape(_TM // 8, 8, 128)


@functools.lru_cache(maxsize=None)
def _make_gram(n_pad: int, d: int):
    grid = (n_pad // _TM, n_pad // _TM)
    return pl.pallas_call(
        _mm_body,
        grid=grid,
        in_specs=[pl.BlockSpec((_TM, d), lambda i, j: (i, 0)),
                  pl.BlockSpec((_TM, d), lambda i, j: (j, 0))],
        out_specs=pl.BlockSpec((_TM // 8, _TM // 128, 8, 128),
                               lambda i, j: (i, j, 0, 0)),
        out_shape=jax.ShapeDtypeStruct(
            (n_pad // 8, n_pad // 128, 8, 128), jnp.float32),
    )


@functools.lru_cache(maxsize=None)
def _make_decoder(n_nodes: int, n_pad: int, n_edges: int):
    info = plsc.get_sparse_core_info()
    nw = info.num_cores * info.num_subcores
    assert n_edges % nw == 0
    per_w = n_edges // nw
    chunk = 0
    for c in range(128, 15, -16):
        if per_w % c == 0:
            chunk = c
            break
    assert chunk > 0
    n_chunks = per_w // chunk
    n_rows = n_pad * n_pad // 16
    np128 = n_pad // 128

    mesh = plsc.VectorSubcoreMesh(core_axis_name="c", subcore_axis_name="s")
    nbuf = 8

    @functools.partial(
        pl.kernel,
        out_type=jax.ShapeDtypeStruct((n_edges,), jnp.float32),
        mesh=mesh,
        compiler_params=pltpu.CompilerParams(needs_layout_passes=False,
                                             use_tc_tiling_on_sc=False),
        scratch_types=[
            pltpu.VMEM((n_chunks, chunk), jnp.int32),
            pltpu.VMEM((n_chunks, chunk), jnp.int32),
            pltpu.VMEM((nbuf, chunk), jnp.int32),
            pltpu.VMEM((nbuf, chunk, 16), jnp.float32),
            pltpu.VMEM((per_w,), jnp.float32),
        ] + [pltpu.SemaphoreType.DMA] * nbuf,
    )
    def decode(gw_hbm, ei_hbm, out_hbm, idx_s, idx_d, row_buf, w_buf,
               out_buf, *sems):
        wid = lax.axis_index("s") * info.num_cores + lax.axis_index("c")
        base = wid * per_w

        pltpu.sync_copy(ei_hbm.at[0, wid], idx_s)
        pltpu.sync_copy(ei_hbm.at[1, wid], idx_d)

        lane = lax.iota(jnp.int32, 16)

        def flat_words(c, g):
            s_vec = idx_s[c, pl.ds(g * _LANES, _LANES)]
            d_vec = idx_d[c, pl.ds(g * _LANES, _LANES)]
            tile = lax.shift_right_logical(s_vec, 3) * np128 + \
                lax.shift_right_logical(d_vec, 7)
            return tile * 1024 + lax.shift_left(s_vec & 7, 7) + (d_vec & 127)

        def rowidx(c, slot):
            def g_body(g, _):
                row_buf[slot, pl.ds(g * _LANES, _LANES)] = \
                    lax.shift_right_logical(flat_words(c, g), 4)
                return 0

            lax.fori_loop(0, chunk // _LANES, g_body, 0)

        def fire(c, slot):
            rowidx(c, slot)
            pltpu.make_async_copy(
                gw_hbm.at[row_buf.at[slot]], w_buf.at[slot],
                sems[slot]).start()

        def drain(slot):
            pltpu.make_async_copy(
                gw_hbm.at[row_buf.at[slot]], w_buf.at[slot],
                sems[slot]).wait()

        def compute(c, slot):
            out_base = c * chunk

            def g_body(g, _):
                e_vec = g * _LANES + lane
                wv = flat_words(c, g) & 15
                val = plsc.load_gather(w_buf.at[slot], [e_vec, wv])
                res = 1.0 / (1.0 + jnp.exp(-val))
                out_buf[pl.ds(out_base + g * _LANES, _LANES)] = res
                return 0

            lax.fori_loop(0, chunk // _LANES, g_body, 0)

        for s in range(nbuf - 1):
            fire(s, s)

        def pipe_body(i, _):
            for j in range(nbuf):
                c = i * nbuf + j

                @pl.when(c + nbuf - 1 < n_chunks)
                def _():
                    fire(c + nbuf - 1, (j + nbuf - 1) % nbuf)

                drain(j)
                compute(c, j)
            return 0

        lax.fori_loop(0, n_chunks // nbuf, pipe_body, 0)
        for j in range(n_chunks % nbuf):
            c = n_chunks - n_chunks % nbuf + j
            drain(c % nbuf)
            compute(c, c % nbuf)

        pltpu.sync_copy(out_buf, out_hbm.at[pl.ds(base, per_w)])

    return decode, nw, n_chunks, chunk, n_rows


def kernel(z, edge_index):
    n_nodes, d = z.shape
    n_edges = edge_index.shape[1]
    n_pad = -(-n_nodes // _TM) * _TM
    decode, nw, n_chunks, chunk, n_rows = _make_decoder(
        n_nodes, n_pad, n_edges)
    z_bf = jnp.pad(z.astype(jnp.bfloat16), ((0, n_pad - n_nodes), (0, 0)))
    gram4d = _make_gram(n_pad, d)(z_bf, z_bf)
    gw = gram4d.reshape(n_rows, 16)
    ei = edge_index.astype(jnp.int32).reshape(2, nw, n_chunks, chunk)
    return decode(gw, ei)


# speedup vs baseline: 408.3578x; 1.6637x over previous
import functools

import jax
import jax.numpy as jnp
from jax import lax
from jax.experimental import pallas as pl
from jax.experimental.pallas import tpu as pltpu
from jax.experimental.pallas import tpu_sc as plsc

_LANES = 16
_TM = 512


def _mm_body(it_ref, jt_ref, a_ref, b_ref, o_ref):
    a = a_ref[...]
    for jj in range(_TM // 128):
        p = lax.dot_general(
            a, b_ref[pl.ds(jj * 128, 128), :], (((1,), (1,)), ((), ())),
            preferred_element_type=jnp.float32)
        o_ref[:, jj] = p.reshape(_TM // 8, 8, 128)


@functools.lru_cache(maxsize=None)
def _make_gram(n_pad: int, d: int):
    nt = n_pad // _TM
    grid = (nt * (nt + 1) // 2,)
    return pl.pallas_call(
        _mm_body,
        grid_spec=pltpu.PrefetchScalarGridSpec(
            num_scalar_prefetch=2,
            grid=grid,
            in_specs=[
                pl.BlockSpec((_TM, d), lambda t, it, jt: (it[t], 0)),
                pl.BlockSpec((_TM, d), lambda t, it, jt: (jt[t], 0)),
            ],
            out_specs=pl.BlockSpec((_TM // 8, _TM // 128, 8, 128),
                                   lambda t, it, jt: (it[t], jt[t], 0, 0)),
        ),
        out_shape=jax.ShapeDtypeStruct(
            (n_pad // 8, n_pad // 128, 8, 128), jnp.float32),
    )


@functools.lru_cache(maxsize=None)
def _make_decoder(n_nodes: int, n_pad: int, n_edges: int):
    info = plsc.get_sparse_core_info()
    nw = info.num_cores * info.num_subcores
    assert n_edges % nw == 0
    per_w = n_edges // nw
    chunk = 0
    for c in range(128, 15, -16):
        if per_w % c == 0:
            chunk = c
            break
    assert chunk > 0
    n_chunks = per_w // chunk
    n_rows = n_pad * n_pad // 16
    np128 = n_pad // 128

    mesh = plsc.VectorSubcoreMesh(core_axis_name="c", subcore_axis_name="s")
    nbuf = 8

    @functools.partial(
        pl.kernel,
        out_type=jax.ShapeDtypeStruct((n_edges,), jnp.float32),
        mesh=mesh,
        compiler_params=pltpu.CompilerParams(needs_layout_passes=False,
                                             use_tc_tiling_on_sc=False),
        scratch_types=[
            pltpu.VMEM((n_chunks, chunk), jnp.int32),
            pltpu.VMEM((n_chunks, chunk), jnp.int32),
            pltpu.VMEM((nbuf, chunk), jnp.int32),
            pltpu.VMEM((nbuf, chunk, 16), jnp.float32),
            pltpu.VMEM((per_w,), jnp.float32),
        ] + [pltpu.SemaphoreType.DMA] * nbuf,
    )
    def decode(gw_hbm, ei_hbm, out_hbm, idx_s, idx_d, row_buf, w_buf,
               out_buf, *sems):
        wid = lax.axis_index("s") * info.num_cores + lax.axis_index("c")
        base = wid * per_w

        pltpu.sync_copy(ei_hbm.at[0, wid], idx_s)
        pltpu.sync_copy(ei_hbm.at[1, wid], idx_d)

        lane = lax.iota(jnp.int32, 16)

        def flat_words(c, g):
            a_vec = idx_s[c, pl.ds(g * _LANES, _LANES)]
            b_vec = idx_d[c, pl.ds(g * _LANES, _LANES)]
            s_vec = jnp.maximum(a_vec, b_vec)
            d_vec = jnp.minimum(a_vec, b_vec)
            tile = lax.shift_right_logical(s_vec, 3) * np128 + \
                lax.shift_right_logical(d_vec, 7)
            return tile * 1024 + lax.shift_left(s_vec & 7, 7) + (d_vec & 127)

        def rowidx(c, slot):
            def g_body(g, _):
                row_buf[slot, pl.ds(g * _LANES, _LANES)] = \
                    lax.shift_right_logical(flat_words(c, g), 4)
                return 0

            lax.fori_loop(0, chunk // _LANES, g_body, 0)

        def fire(c, slot):
            rowidx(c, slot)
            pltpu.make_async_copy(
                gw_hbm.at[row_buf.at[slot]], w_buf.at[slot],
                sems[slot]).start()

        def drain(slot):
            pltpu.make_async_copy(
                gw_hbm.at[row_buf.at[slot]], w_buf.at[slot],
                sems[slot]).wait()

        def compute(c, slot):
            out_base = c * chunk

            def g_body(g, _):
                e_vec = g * _LANES + lane
                wv = flat_words(c, g) & 15
                val = plsc.load_gather(w_buf.at[slot], [e_vec, wv])
                res = 1.0 / (1.0 + jnp.exp(-val))
                out_buf[pl.ds(out_base + g * _LANES, _LANES)] = res
                return 0

            lax.fori_loop(0, chunk // _LANES, g_body, 0)

        for s in range(nbuf - 1):
            fire(s, s)

        def pipe_body(i, _):
            for j in range(nbuf):
                c = i * nbuf + j

                @pl.when(c + nbuf - 1 < n_chunks)
                def _():
                    fire(c + nbuf - 1, (j + nbuf - 1) % nbuf)

                drain(j)
                compute(c, j)
            return 0

        lax.fori_loop(0, n_chunks // nbuf, pipe_body, 0)
        for j in range(n_chunks % nbuf):
            c = n_chunks - n_chunks % nbuf + j
            drain(c % nbuf)
            compute(c, c % nbuf)

        pltpu.sync_copy(out_buf, out_hbm.at[pl.ds(base, per_w)])

    return decode, nw, n_chunks, chunk, n_rows


def kernel(z, edge_index):
    n_nodes, d = z.shape
    n_edges = edge_index.shape[1]
    n_pad = -(-n_nodes // _TM) * _TM
    decode, nw, n_chunks, chunk, n_rows = _make_decoder(
        n_nodes, n_pad, n_edges)
    z_bf = jnp.pad(z.astype(jnp.bfloat16), ((0, n_pad - n_nodes), (0, 0)))
    nt = n_pad // _TM
    it = jnp.array([i for i in range(nt) for _ in range(i + 1)], jnp.int32)
    jt = jnp.array([j for i in range(nt) for j in range(i + 1)], jnp.int32)
    gram4d = _make_gram(n_pad, d)(it, jt, z_bf, z_bf)
    gw = gram4d.reshape(n_rows, 16)
    ei = edge_index.astype(jnp.int32).reshape(2, nw, n_chunks, chunk)
    return decode(gw, ei)


# speedup vs baseline: 427.6885x; 1.0473x over previous
import functools

import jax
import jax.numpy as jnp
from jax import lax
from jax.experimental import pallas as pl
from jax.experimental.pallas import tpu as pltpu
from jax.experimental.pallas import tpu_sc as plsc

_LANES = 16
_TM = 512


def _mm_body(it_ref, jt_ref, a_ref, b_ref, o_ref):
    def _bf16_bits(x):
        u = lax.bitcast_convert_type(x, jnp.int32)
        return lax.shift_right_logical(
            u + 0x7FFF + (lax.shift_right_logical(u, 16) & 1), 16)

    a = a_ref[...]
    for jj in range(_TM // 256):
        p = lax.dot_general(
            a, b_ref[pl.ds(jj * 256, 256), :], (((1,), (1,)), ((), ())),
            preferred_element_type=jnp.float32)
        w = _bf16_bits(p[:, :128]) | lax.shift_left(
            _bf16_bits(p[:, 128:]), 16)
        o_ref[:, jj] = w.reshape(_TM // 8, 8, 128)


@functools.lru_cache(maxsize=None)
def _make_gram(n_pad: int, d: int):
    nt = n_pad // _TM
    grid = (nt * (nt + 1) // 2,)
    return pl.pallas_call(
        _mm_body,
        grid_spec=pltpu.PrefetchScalarGridSpec(
            num_scalar_prefetch=2,
            grid=grid,
            in_specs=[
                pl.BlockSpec((_TM, d), lambda t, it, jt: (it[t], 0)),
                pl.BlockSpec((_TM, d), lambda t, it, jt: (jt[t], 0)),
            ],
            out_specs=pl.BlockSpec((_TM // 8, _TM // 256, 8, 128),
                                   lambda t, it, jt: (it[t], jt[t], 0, 0)),
        ),
        out_shape=jax.ShapeDtypeStruct(
            (n_pad // 8, n_pad // 256, 8, 128), jnp.int32),
    )


@functools.lru_cache(maxsize=None)
def _make_decoder(n_nodes: int, n_pad: int, n_edges: int):
    info = plsc.get_sparse_core_info()
    nw = info.num_cores * info.num_subcores
    assert n_edges % nw == 0
    per_w = n_edges // nw
    chunk = 0
    for c in range(128, 15, -16):
        if per_w % c == 0:
            chunk = c
            break
    assert chunk > 0
    n_chunks = per_w // chunk
    n_rows = n_pad * n_pad // 32
    np128 = n_pad // 256

    mesh = plsc.VectorSubcoreMesh(core_axis_name="c", subcore_axis_name="s")
    nbuf = 8

    @functools.partial(
        pl.kernel,
        out_type=jax.ShapeDtypeStruct((n_edges,), jnp.float32),
        mesh=mesh,
        compiler_params=pltpu.CompilerParams(needs_layout_passes=False,
                                             use_tc_tiling_on_sc=False),
        scratch_types=[
            pltpu.VMEM((n_chunks, chunk), jnp.int32),
            pltpu.VMEM((n_chunks, chunk), jnp.int32),
            pltpu.VMEM((nbuf, chunk), jnp.int32),
            pltpu.VMEM((nbuf, chunk, 16), jnp.int32),
            pltpu.VMEM((per_w,), jnp.float32),
        ] + [pltpu.SemaphoreType.DMA] * nbuf,
    )
    def decode(gw_hbm, ei_hbm, out_hbm, idx_s, idx_d, row_buf, w_buf,
               out_buf, *sems):
        wid = lax.axis_index("s") * info.num_cores + lax.axis_index("c")
        base = wid * per_w

        pltpu.sync_copy(ei_hbm.at[0, wid], idx_s)
        pltpu.sync_copy(ei_hbm.at[1, wid], idx_d)

        lane = lax.iota(jnp.int32, 16)

        def flat_words(c, g):
            a_vec = idx_s[c, pl.ds(g * _LANES, _LANES)]
            b_vec = idx_d[c, pl.ds(g * _LANES, _LANES)]
            s_vec = jnp.maximum(a_vec, b_vec)
            d_vec = jnp.minimum(a_vec, b_vec)
            tile = lax.shift_right_logical(s_vec, 3) * np128 + \
                lax.shift_right_logical(d_vec, 8)
            return (tile * 1024 + lax.shift_left(s_vec & 7, 7) +
                    (d_vec & 127), lax.shift_right_logical(d_vec, 7) & 1)

        def rowidx(c, slot):
            def g_body(g, _):
                row_buf[slot, pl.ds(g * _LANES, _LANES)] = \
                    lax.shift_right_logical(flat_words(c, g)[0], 4)
                return 0

            lax.fori_loop(0, chunk // _LANES, g_body, 0)

        def fire(c, slot):
            rowidx(c, slot)
            pltpu.make_async_copy(
                gw_hbm.at[row_buf.at[slot]], w_buf.at[slot],
                sems[slot]).start()

        def drain(slot):
            pltpu.make_async_copy(
                gw_hbm.at[row_buf.at[slot]], w_buf.at[slot],
                sems[slot]).wait()

        def compute(c, slot):
            out_base = c * chunk

            def g_body(g, _):
                e_vec = g * _LANES + lane
                words, parity = flat_words(c, g)
                w = plsc.load_gather(w_buf.at[slot], [e_vec, words & 15])
                lo, hi = plsc.unpack(
                    plsc.bitcast(w, jnp.bfloat16),
                    format=plsc.PackFormat.INTERLEAVED,
                    preferred_element_type=jnp.float32)
                val = jnp.where(parity == 1, hi, lo)
                res = 1.0 / (1.0 + jnp.exp(-val))
                out_buf[pl.ds(out_base + g * _LANES, _LANES)] = res
                return 0

            lax.fori_loop(0, chunk // _LANES, g_body, 0)

        for s in range(nbuf - 1):
            fire(s, s)

        def pipe_body(i, _):
            for j in range(nbuf):
                c = i * nbuf + j

                @pl.when(c + nbuf - 1 < n_chunks)
                def _():
                    fire(c + nbuf - 1, (j + nbuf - 1) % nbuf)

                drain(j)
                compute(c, j)
            return 0

        lax.fori_loop(0, n_chunks // nbuf, pipe_body, 0)
        for j in range(n_chunks % nbuf):
            c = n_chunks - n_chunks % nbuf + j
            drain(c % nbuf)
            compute(c, c % nbuf)

        pltpu.sync_copy(out_buf, out_hbm.at[pl.ds(base, per_w)])

    return decode, nw, n_chunks, chunk, n_rows


def kernel(z, edge_index):
    n_nodes, d = z.shape
    n_edges = edge_index.shape[1]
    n_pad = -(-n_nodes // _TM) * _TM
    decode, nw, n_chunks, chunk, n_rows = _make_decoder(
        n_nodes, n_pad, n_edges)
    z_bf = jnp.pad(z.astype(jnp.bfloat16), ((0, n_pad - n_nodes), (0, 0)))
    nt = n_pad // _TM
    it = jnp.array([i for i in range(nt) for _ in range(i + 1)], jnp.int32)
    jt = jnp.array([j for i in range(nt) for j in range(i + 1)], jnp.int32)
    gram4d = _make_gram(n_pad, d)(it, jt, z_bf, z_bf)
    gw = gram4d.reshape(n_rows, 16)
    ei = edge_index.astype(jnp.int32).reshape(2, nw, n_chunks, chunk)
    return decode(gw, ei)


# speedup vs baseline: 762.1048x; 1.7819x over previous
import functools

import jax
import jax.numpy as jnp
from jax import lax
from jax.experimental import pallas as pl
from jax.experimental.pallas import tpu as pltpu
from jax.experimental.pallas import tpu_sc as plsc

_LANES = 16
_TM = 1024


def _mm_body(it_ref, jt_ref, a_ref, b_ref, o_ref):
    def _bf16_bits(x):
        u = lax.bitcast_convert_type(x, jnp.int32)
        return lax.shift_right_logical(u + 0x8000, 16)

    a = a_ref[...]
    for jj in range(_TM // 256):
        p = lax.dot_general(
            a, b_ref[pl.ds(jj * 256, 256), :], (((1,), (1,)), ((), ())),
            preferred_element_type=jnp.float32)
        w = _bf16_bits(p[:, :128]) | lax.shift_left(
            _bf16_bits(p[:, 128:]), 16)
        o_ref[:, jj] = w.reshape(_TM // 8, 8, 128)


@functools.lru_cache(maxsize=None)
def _make_gram(n_pad: int, d: int):
    nt = n_pad // _TM
    grid = (nt * (nt + 1) // 2,)
    return pl.pallas_call(
        _mm_body,
        grid_spec=pltpu.PrefetchScalarGridSpec(
            num_scalar_prefetch=2,
            grid=grid,
            in_specs=[
                pl.BlockSpec((_TM, d), lambda t, it, jt: (it[t], 0)),
                pl.BlockSpec((_TM, d), lambda t, it, jt: (jt[t], 0)),
            ],
            out_specs=pl.BlockSpec((_TM // 8, _TM // 256, 8, 128),
                                   lambda t, it, jt: (it[t], jt[t], 0, 0)),
        ),
        out_shape=jax.ShapeDtypeStruct(
            (n_pad // 8, n_pad // 256, 8, 128), jnp.int32),
    )


@functools.lru_cache(maxsize=None)
def _make_decoder(n_nodes: int, n_pad: int, n_edges: int):
    info = plsc.get_sparse_core_info()
    nw = info.num_cores * info.num_subcores
    assert n_edges % nw == 0
    per_w = n_edges // nw
    chunk = 0
    for c in range(128, 15, -16):
        if per_w % c == 0:
            chunk = c
            break
    assert chunk > 0
    n_chunks = per_w // chunk
    n_rows = n_pad * n_pad // 32
    np128 = n_pad // 256

    mesh = plsc.VectorSubcoreMesh(core_axis_name="c", subcore_axis_name="s")
    nbuf = 8

    @functools.partial(
        pl.kernel,
        out_type=jax.ShapeDtypeStruct((n_edges,), jnp.float32),
        mesh=mesh,
        compiler_params=pltpu.CompilerParams(needs_layout_passes=False,
                                             use_tc_tiling_on_sc=False),
        scratch_types=[
            pltpu.VMEM((n_chunks, chunk), jnp.int32),
            pltpu.VMEM((n_chunks, chunk), jnp.int32),
            pltpu.VMEM((nbuf, chunk), jnp.int32),
            pltpu.VMEM((nbuf, chunk, 16), jnp.int32),
            pltpu.VMEM((per_w,), jnp.float32),
        ] + [pltpu.SemaphoreType.DMA] * nbuf,
    )
    def decode(gw_hbm, ei_hbm, out_hbm, idx_s, idx_d, row_buf, w_buf,
               out_buf, *sems):
        wid = lax.axis_index("s") * info.num_cores + lax.axis_index("c")
        base = wid * per_w

        pltpu.sync_copy(ei_hbm.at[0, wid], idx_s)
        pltpu.sync_copy(ei_hbm.at[1, wid], idx_d)

        lane = lax.iota(jnp.int32, 16)

        def flat_words(c, g):
            a_vec = idx_s[c, pl.ds(g * _LANES, _LANES)]
            b_vec = idx_d[c, pl.ds(g * _LANES, _LANES)]
            s_vec = jnp.maximum(a_vec, b_vec)
            d_vec = jnp.minimum(a_vec, b_vec)
            tile = lax.shift_right_logical(s_vec, 3) * np128 + \
                lax.shift_right_logical(d_vec, 8)
            return (tile * 1024 + lax.shift_left(s_vec & 7, 7) +
                    (d_vec & 127), lax.shift_right_logical(d_vec, 7) & 1)

        def rowidx(c, slot):
            def g_body(g, _):
                row_buf[slot, pl.ds(g * _LANES, _LANES)] = \
                    lax.shift_right_logical(flat_words(c, g)[0], 4)
                return 0

            lax.fori_loop(0, chunk // _LANES, g_body, 0)

        def fire(c, slot):
            rowidx(c, slot)
            pltpu.make_async_copy(
                gw_hbm.at[row_buf.at[slot]], w_buf.at[slot],
                sems[slot]).start()

        def drain(slot):
            pltpu.make_async_copy(
                gw_hbm.at[row_buf.at[slot]], w_buf.at[slot],
                sems[slot]).wait()

        def compute(c, slot):
            out_base = c * chunk

            def g_body(g, _):
                e_vec = g * _LANES + lane
                words, parity = flat_words(c, g)
                w = plsc.load_gather(w_buf.at[slot], [e_vec, words & 15])
                lo, hi = plsc.unpack(
                    plsc.bitcast(w, jnp.bfloat16),
                    format=plsc.PackFormat.INTERLEAVED,
                    preferred_element_type=jnp.float32)
                val = jnp.where(parity == 1, hi, lo)
                res = 1.0 / (1.0 + jnp.exp(-val))
                out_buf[pl.ds(out_base + g * _LANES, _LANES)] = res
                return 0

            lax.fori_loop(0, chunk // _LANES, g_body, 0)

        for s in range(nbuf - 1):
            fire(s, s)

        def pipe_body(i, _):
            for j in range(nbuf):
                c = i * nbuf + j

                @pl.when(c + nbuf - 1 < n_chunks)
                def _():
                    fire(c + nbuf - 1, (j + nbuf - 1) % nbuf)

                drain(j)
                compute(c, j)
            return 0

        lax.fori_loop(0, n_chunks // nbuf, pipe_body, 0)
        for j in range(n_chunks % nbuf):
            c = n_chunks - n_chunks % nbuf + j
            drain(c % nbuf)
            compute(c, c % nbuf)

        pltpu.sync_copy(out_buf, out_hbm.at[pl.ds(base, per_w)])

    return decode, nw, n_chunks, chunk, n_rows


def kernel(z, edge_index):
    n_nodes, d = z.shape
    n_edges = edge_index.shape[1]
    n_pad = -(-n_nodes // _TM) * _TM
    decode, nw, n_chunks, chunk, n_rows = _make_decoder(
        n_nodes, n_pad, n_edges)
    z_bf = jnp.pad(z.astype(jnp.bfloat16), ((0, n_pad - n_nodes), (0, 0)))
    nt = n_pad // _TM
    it = jnp.array([i for i in range(nt) for _ in range(i + 1)], jnp.int32)
    jt = jnp.array([j for i in range(nt) for j in range(i + 1)], jnp.int32)
    gram4d = _make_gram(n_pad, d)(it, jt, z_bf, z_bf)
    gw = gram4d.reshape(n_rows, 16)
    ei = edge_index.astype(jnp.int32).reshape(2, nw, n_chunks, chunk)
    return decode(gw, ei)


# speedup vs baseline: 926.8294x; 1.2161x over previous
import functools

import jax
import jax.numpy as jnp
from jax import lax
from jax.experimental import pallas as pl
from jax.experimental.pallas import tpu as pltpu
from jax.experimental.pallas import tpu_sc as plsc

_LANES = 16
_TM = 2048


def _mm_body(it_ref, jt_ref, a_ref, b_ref, o_ref):
    def _bf16_bits(x):
        u = lax.bitcast_convert_type(x, jnp.int32)
        return lax.shift_right_logical(u + 0x8000, 16)

    a = a_ref[...]
    for jj in range(_TM // 256):
        p = lax.dot_general(
            a, b_ref[pl.ds(jj * 256, 256), :], (((1,), (1,)), ((), ())),
            preferred_element_type=jnp.float32)
        w = _bf16_bits(p[:, :128]) | lax.shift_left(
            _bf16_bits(p[:, 128:]), 16)
        o_ref[:, jj] = w.reshape(_TM // 8, 8, 128)


@functools.lru_cache(maxsize=None)
def _make_gram(n_pad: int, d: int):
    nt = n_pad // _TM
    grid = (nt * (nt + 1) // 2,)
    return pl.pallas_call(
        _mm_body,
        grid_spec=pltpu.PrefetchScalarGridSpec(
            num_scalar_prefetch=2,
            grid=grid,
            in_specs=[
                pl.BlockSpec((_TM, d), lambda t, it, jt: (it[t], 0)),
                pl.BlockSpec((_TM, d), lambda t, it, jt: (jt[t], 0)),
            ],
            out_specs=pl.BlockSpec((_TM // 8, _TM // 256, 8, 128),
                                   lambda t, it, jt: (it[t], jt[t], 0, 0)),
        ),
        out_shape=jax.ShapeDtypeStruct(
            (n_pad // 8, n_pad // 256, 8, 128), jnp.int32),
    )


@functools.lru_cache(maxsize=None)
def _make_decoder(n_nodes: int, n_pad: int, n_edges: int):
    info = plsc.get_sparse_core_info()
    nw = info.num_cores * info.num_subcores
    assert n_edges % nw == 0
    per_w = n_edges // nw
    chunk = 0
    for c in range(128, 15, -16):
        if per_w % c == 0:
            chunk = c
            break
    assert chunk > 0
    n_chunks = per_w // chunk
    n_rows = n_pad * n_pad // 32
    np128 = n_pad // 256

    mesh = plsc.VectorSubcoreMesh(core_axis_name="c", subcore_axis_name="s")
    nbuf = 8

    @functools.partial(
        pl.kernel,
        out_type=jax.ShapeDtypeStruct((n_edges,), jnp.float32),
        mesh=mesh,
        compiler_params=pltpu.CompilerParams(needs_layout_passes=False,
                                             use_tc_tiling_on_sc=False),
        scratch_types=[
            pltpu.VMEM((n_chunks, chunk), jnp.int32),
            pltpu.VMEM((n_chunks, chunk), jnp.int32),
            pltpu.VMEM((nbuf, chunk), jnp.int32),
            pltpu.VMEM((nbuf, chunk, 16), jnp.int32),
            pltpu.VMEM((per_w,), jnp.float32),
        ] + [pltpu.SemaphoreType.DMA] * nbuf,
    )
    def decode(gw_hbm, ei_hbm, out_hbm, idx_s, idx_d, row_buf, w_buf,
               out_buf, *sems):
        wid = lax.axis_index("s") * info.num_cores + lax.axis_index("c")
        base = wid * per_w

        pltpu.sync_copy(ei_hbm.at[0, wid], idx_s)
        pltpu.sync_copy(ei_hbm.at[1, wid], idx_d)

        lane = lax.iota(jnp.int32, 16)

        def flat_words(c, g):
            a_vec = idx_s[c, pl.ds(g * _LANES, _LANES)]
            b_vec = idx_d[c, pl.ds(g * _LANES, _LANES)]
            s_vec = jnp.maximum(a_vec, b_vec)
            d_vec = jnp.minimum(a_vec, b_vec)
            tile = lax.shift_right_logical(s_vec, 3) * np128 + \
                lax.shift_right_logical(d_vec, 8)
            return (tile * 1024 + lax.shift_left(s_vec & 7, 7) +
                    (d_vec & 127), lax.shift_right_logical(d_vec, 7) & 1)

        def rowidx(c, slot):
            def g_body(g, _):
                row_buf[slot, pl.ds(g * _LANES, _LANES)] = \
                    lax.shift_right_logical(flat_words(c, g)[0], 4)
                return 0

            lax.fori_loop(0, chunk // _LANES, g_body, 0)

        def fire(c, slot):
            rowidx(c, slot)
            pltpu.make_async_copy(
                gw_hbm.at[row_buf.at[slot]], w_buf.at[slot],
                sems[slot]).start()

        def drain(slot):
            pltpu.make_async_copy(
                gw_hbm.at[row_buf.at[slot]], w_buf.at[slot],
                sems[slot]).wait()

        def compute(c, slot):
            out_base = c * chunk

            def g_body(g, _):
                e_vec = g * _LANES + lane
                words, parity = flat_words(c, g)
                w = plsc.load_gather(w_buf.at[slot], [e_vec, words & 15])
                lo, hi = plsc.unpack(
                    plsc.bitcast(w, jnp.bfloat16),
                    format=plsc.PackFormat.INTERLEAVED,
                    preferred_element_type=jnp.float32)
                val = jnp.where(parity == 1, hi, lo)
                res = 1.0 / (1.0 + jnp.exp(-val))
                out_buf[pl.ds(out_base + g * _LANES, _LANES)] = res
                return 0

            lax.fori_loop(0, chunk // _LANES, g_body, 0)

        for s in range(nbuf - 1):
            fire(s, s)

        def pipe_body(i, _):
            for j in range(nbuf):
                c = i * nbuf + j

                @pl.when(c + nbuf - 1 < n_chunks)
                def _():
                    fire(c + nbuf - 1, (j + nbuf - 1) % nbuf)

                drain(j)
                compute(c, j)
            return 0

        lax.fori_loop(0, n_chunks // nbuf, pipe_body, 0)
        for j in range(n_chunks % nbuf):
            c = n_chunks - n_chunks % nbuf + j
            drain(c % nbuf)
            compute(c, c % nbuf)

        pltpu.sync_copy(out_buf, out_hbm.at[pl.ds(base, per_w)])

    return decode, nw, n_chunks, chunk, n_rows


def kernel(z, edge_index):
    n_nodes, d = z.shape
    n_edges = edge_index.shape[1]
    n_pad = -(-n_nodes // _TM) * _TM
    decode, nw, n_chunks, chunk, n_rows = _make_decoder(
        n_nodes, n_pad, n_edges)
    z_bf = jnp.pad(z.astype(jnp.bfloat16), ((0, n_pad - n_nodes), (0, 0)))
    nt = n_pad // _TM
    it = jnp.array([i for i in range(nt) for _ in range(i + 1)], jnp.int32)
    jt = jnp.array([j for i in range(nt) for j in range(i + 1)], jnp.int32)
    gram4d = _make_gram(n_pad, d)(it, jt, z_bf, z_bf)
    gw = gram4d.reshape(n_rows, 16)
    ei = edge_index.astype(jnp.int32).reshape(2, nw, n_chunks, chunk)
    return decode(gw, ei)
